# Initial kernel scaffold; baseline (speedup 1.0000x reference)
import jax
import jax.numpy as jnp
from jax.experimental import pallas as pl


def kernel(u, v, t, event, h0, W1_0, W2_0, W1_1, W2_1):
    raise NotImplementedError("write your pallas kernel here")


# speedup vs baseline: 3.7449x; 3.7449x over previous
import functools

import jax
import jax.numpy as jnp
from jax import lax
from jax.experimental import pallas as pl
from jax.experimental.pallas import tpu as pltpu
from jax.experimental.pallas import tpu_sc as plsc

EPS = 1e-5
NC = 2
NS = 16
LN = 128
KC = 2
CB = 32


def _rup(x, m):
    return ((x + m - 1) // m) * m


def _sc_aggregate(htab, dst2, srcp, gv2, hcol, zrow, zsc,
                  n2, n_pad, n_scalar, ep_rows):
    rpt = n_pad // NS
    t_rows = ep_rows // NS
    n_pass = n2 // 2
    mesh = plsc.VectorSubcoreMesh(core_axis_name="c", subcore_axis_name="s")

    @functools.partial(
        pl.kernel,
        out_type=(
            jax.ShapeDtypeStruct((n2, n_pad, CB), jnp.float32),
            jax.ShapeDtypeStruct((n_scalar, n_pad), jnp.float32),
        ),
        mesh=mesh,
        compiler_params=pltpu.CompilerParams(use_tc_tiling_on_sc=False),
        scratch_types=(
            pltpu.VMEM_SHARED((n_pad, CB), jnp.float32),
            pltpu.VMEM_SHARED((n_pad,), jnp.float32),
            pltpu.VMEM((KC, LN), jnp.int32),
            pltpu.VMEM((KC, LN), jnp.int32),
            pltpu.VMEM((KC, LN), jnp.int32),
            pltpu.VMEM((KC * LN, CB), jnp.float32),
            pltpu.VMEM((KC, LN), jnp.float32),
            pltpu.VMEM((KC, LN), jnp.float32),
            pltpu.SemaphoreType.DMA,
            pltpu.SemaphoreType.DMA,
        ),
    )
    def body(htab_r, dst_r, srcp_r, gv_r, hcol_r, zrow_r, zsc_r,
             outh_r, outs_r,
             acc, accs, dbuf, sbuf, rbuf, rows, gbuf, cbuf, sem, sem2):
        c = lax.axis_index("c")
        s = lax.axis_index("s")
        for p in range(n_pass):
            blk = p * 2 + c
            pltpu.sync_copy(zrow_r, acc.at[pl.ds(s * rpt, rpt)])
            if p == 0 and n_scalar >= 1:
                @pl.when(c < n_scalar)
                def _():
                    pltpu.sync_copy(zsc_r, accs.at[pl.ds(s * rpt, rpt)])
            plsc.subcore_barrier()

            def chunk(i, carry):
                rb = s * t_rows + i * KC
                pltpu.sync_copy(dst_r.at[pl.ds(rb, KC)], dbuf)
                pltpu.sync_copy(srcp_r.at[blk, pl.ds(rb, KC)], sbuf)
                cps = [
                    pltpu.async_copy(
                        htab_r.at[sbuf.at[j]],
                        rows.at[pl.ds(j * LN, LN)], sem)
                    for j in range(KC)
                ]
                for j in range(KC):
                    cps[j].wait()
                    pltpu.sync_copy(rows.at[pl.ds(j * LN, LN)],
                                    acc.at[dbuf.at[j]], add=True)
                if p == 0 and n_scalar >= 1:
                    @pl.when(c == 0)
                    def _():
                        pltpu.sync_copy(gv_r.at[pl.ds(rb, KC)], gbuf)
                        for j in range(KC):
                            pltpu.sync_copy(gbuf.at[j],
                                            accs.at[dbuf.at[j]], add=True)
                if p == 0 and n_scalar == 2:
                    @pl.when(c == 1)
                    def _():
                        pltpu.sync_copy(srcp_r.at[n2, pl.ds(rb, KC)], rbuf)
                        cc = [
                            pltpu.async_copy(hcol_r.at[rbuf.at[j]],
                                             cbuf.at[j], sem2)
                            for j in range(KC)
                        ]
                        for j in range(KC):
                            cc[j].wait()
                            pltpu.sync_copy(cbuf.at[j],
                                            accs.at[dbuf.at[j]], add=True)
                return carry

            lax.fori_loop(0, t_rows // KC, chunk, 0)
            plsc.subcore_barrier()
            pltpu.sync_copy(acc.at[pl.ds(s * rpt, rpt)],
                            outh_r.at[blk, pl.ds(s * rpt, rpt)])
            if p == 0 and n_scalar >= 1:
                @pl.when(c < n_scalar)
                def _():
                    pltpu.sync_copy(accs.at[pl.ds(s * rpt, rpt)],
                                    outs_r.at[c, pl.ds(s * rpt, rpt)])
            plsc.subcore_barrier()

    return body(htab, dst2, srcp, gv2, hcol, zrow, zsc)


def _sc_layer(h, dst2, srcr, gv2, n_pad, ep_rows):
    n, d = h.shape
    n2 = max(2, (d // CB) - ((d // CB) % 2))
    rem = d - n2 * CB
    n_scalar = 1 + (1 if rem else 0)
    htab = (h[:, :n2 * CB].reshape(n, n2, CB).transpose(1, 0, 2)
            .reshape(n2 * n, CB))
    offs = [srcr + b * n for b in range(n2)]
    if rem:
        offs.append(srcr)
        hcol = h[:, n2 * CB]
    else:
        hcol = jnp.zeros((8,), jnp.float32)
    srcp = jnp.stack(offs).reshape(len(offs), ep_rows, LN)
    rpt = n_pad // NS
    zrow = jnp.zeros((rpt, CB), jnp.float32)
    zsc = jnp.zeros((rpt,), jnp.float32)
    outh, outs = _sc_aggregate(htab, dst2, srcp, gv2, hcol, zrow, zsc,
                               n2, n_pad, n_scalar, ep_rows)
    hag = outh.transpose(1, 0, 2).reshape(n_pad, n2 * CB)[:n]
    cols = [hag]
    if rem:
        cols.append(outs[1, :n, None])
    cols.append(outs[0, :n, None])
    return jnp.concatenate(cols, axis=1)


_TC_R = 512


def _stats_body(agg_ref, out_ref):
    i = pl.program_id(0)
    x = agg_ref[...]
    s1 = jnp.sum(x, axis=0)
    s2 = jnp.sum(x * x, axis=0)
    part = jnp.concatenate(
        [s1[None], s2[None], jnp.zeros((6, x.shape[1]), jnp.float32)], axis=0)

    @pl.when(i == 0)
    def _():
        out_ref[...] = part

    @pl.when(i > 0)
    def _():
        out_ref[...] += part


def _make_transform_body(n):
    inv_n = 1.0 / n

    def body(agg_ref, h_ref, st_ref, w1_ref, w2a_ref, w2b_ref, o_ref):
        st = st_ref[...]
        mean = st[0:1] * inv_n
        var = st[1:2] * inv_n - mean * mean
        inv = lax.rsqrt(var + EPS)
        bn = (agg_ref[...] - mean) * inv
        r = jnp.maximum(
            jnp.dot(bn, w1_ref[...], preferred_element_type=jnp.float32), 0.0)
        o_ref[...] = (
            jnp.dot(h_ref[...], w2a_ref[...],
                    preferred_element_type=jnp.float32)
            + jnp.dot(r, w2b_ref[...], preferred_element_type=jnp.float32))

    return body


def _tc_dense(agg, h, W1, W2):
    n, da = agg.shape
    di = h.shape[1]
    do = W2.shape[0]
    dap, dip, dop = _rup(da, 128), _rup(di, 128), _rup(do, 128)
    nt = _rup(n, _TC_R)
    aggp = jnp.zeros((nt, dap), jnp.float32).at[:n, :da].set(agg)
    hp = jnp.zeros((nt, dip), jnp.float32).at[:n, :di].set(h)
    w1t = jnp.zeros((dap, dap), jnp.float32).at[:da, :da].set(W1.T)
    w2at = jnp.zeros((dip, dop), jnp.float32).at[:di, :do].set(W2[:, :di].T)
    w2bt = jnp.zeros((dap, dop), jnp.float32).at[:da, :do].set(W2[:, di:].T)
    nb = nt // _TC_R
    stats = pl.pallas_call(
        _stats_body,
        grid=(nb,),
        in_specs=[pl.BlockSpec((_TC_R, dap), lambda i: (i, 0))],
        out_specs=pl.BlockSpec((8, dap), lambda i: (0, 0)),
        out_shape=jax.ShapeDtypeStruct((8, dap), jnp.float32),
    )(aggp)
    out = pl.pallas_call(
        _make_transform_body(n),
        grid=(nb,),
        in_specs=[
            pl.BlockSpec((_TC_R, dap), lambda i: (i, 0)),
            pl.BlockSpec((_TC_R, dip), lambda i: (i, 0)),
            pl.BlockSpec((8, dap), lambda i: (0, 0)),
            pl.BlockSpec((dap, dap), lambda i: (0, 0)),
            pl.BlockSpec((dip, dop), lambda i: (0, 0)),
            pl.BlockSpec((dap, dop), lambda i: (0, 0)),
        ],
        out_specs=pl.BlockSpec((_TC_R, dop), lambda i: (i, 0)),
        out_shape=jax.ShapeDtypeStruct((nt, dop), jnp.float32),
    )(aggp, hp, stats, w1t, w2at, w2bt)
    return out[:n, :do]


def kernel(u, v, t, event, h0, W1_0, W2_0, W1_1, W2_1):
    e = u.shape[0]
    n = h0.shape[0]
    tfirst = t[0]
    tlast = t[event - 1]
    g = (tlast - t) / (1.0 + tlast - tfirst)

    per = NS * KC * LN
    ep = _rup(2 * e, per)
    ep_rows = ep // LN
    n_pad = _rup(n, NS * 128)
    if ep > 2 * e and n_pad == n:
        n_pad += NS * 128
    pad = ep - 2 * e
    dst = jnp.concatenate(
        [u.astype(jnp.int32), v.astype(jnp.int32),
         jnp.full((pad,), n_pad - 1, jnp.int32)])
    srcr = jnp.concatenate(
        [v.astype(jnp.int32), u.astype(jnp.int32),
         jnp.zeros((pad,), jnp.int32)])
    gv2 = jnp.concatenate([g, g, jnp.zeros((pad,), jnp.float32)])
    dst2 = dst.reshape(ep_rows, LN)
    gv2 = gv2.reshape(ep_rows, LN)

    h = h0
    hs = [h]
    for W1, W2 in ((W1_0, W2_0), (W1_1, W2_1)):
        agg = _sc_layer(h, dst2, srcr, gv2, n_pad, ep_rows)
        h = _tc_dense(agg, h, W1, W2)
        hs.append(h)
    return tuple(hs)


# speedup vs baseline: 4.9395x; 1.3190x over previous
import functools

import jax
import jax.numpy as jnp
from jax import lax
from jax.experimental import pallas as pl
from jax.experimental.pallas import tpu as pltpu
from jax.experimental.pallas import tpu_sc as plsc

EPS = 1e-5
NC = 2
NS = 16
LN = 128
KC = 2
CB = 32


def _rup(x, m):
    return ((x + m - 1) // m) * m


def _sc_aggregate(htab, dst2, srcp, gv2, hcol, zrow, zsc,
                  n2, n_pad, n_scalar, ep_rows):
    rpt = n_pad // NS
    t_rows = ep_rows // NS
    n_pass = n2 // 2
    n_chunks = t_rows // KC
    assert n_chunks % 2 == 0
    mesh = plsc.VectorSubcoreMesh(core_axis_name="c", subcore_axis_name="s")

    @functools.partial(
        pl.kernel,
        out_type=(
            jax.ShapeDtypeStruct((n2, n_pad, CB), jnp.float32),
            jax.ShapeDtypeStruct((n_scalar, n_pad), jnp.float32),
        ),
        mesh=mesh,
        compiler_params=pltpu.CompilerParams(use_tc_tiling_on_sc=False),
        scratch_types=(
            pltpu.VMEM_SHARED((n_pad, CB), jnp.float32),
            pltpu.VMEM_SHARED((n_pad,), jnp.float32),
            pltpu.VMEM((2, KC, LN), jnp.int32),
            pltpu.VMEM((2, KC, LN), jnp.int32),
            pltpu.VMEM((2, KC, LN), jnp.int32),
            pltpu.VMEM((2, KC * LN, CB), jnp.float32),
            pltpu.VMEM((2, KC, LN), jnp.float32),
            pltpu.VMEM((2, KC * LN), jnp.float32),
            pltpu.SemaphoreType.DMA,
            pltpu.SemaphoreType.DMA,
            pltpu.SemaphoreType.DMA,
            pltpu.SemaphoreType.DMA,
        ),
    )
    def body(htab_r, dst_r, srcp_r, gv_r, hcol_r, zrow_r, zsc_r,
             outh_r, outs_r,
             acc, accs, dbuf, sbuf, rbuf, rows, gbuf, cbuf,
             semg0, semg1, semc0, semc1):
        c = lax.axis_index("c")
        s = lax.axis_index("s")
        semg = (semg0, semg1)
        semc = (semc0, semc1)
        for p in range(n_pass):
            blk = p * 2 + c

            def load(i, b):
                rb = s * t_rows + i * KC
                pltpu.sync_copy(dst_r.at[pl.ds(rb, KC)], dbuf.at[b])
                pltpu.sync_copy(srcp_r.at[blk, pl.ds(rb, KC)], sbuf.at[b])
                for j in range(KC):
                    pltpu.async_copy(
                        htab_r.at[sbuf.at[b, j]],
                        rows.at[b, pl.ds(j * LN, LN)], semg[b])
                if p == 0 and n_scalar >= 1:
                    @pl.when(c == 0)
                    def _():
                        pltpu.sync_copy(gv_r.at[pl.ds(rb, KC)], gbuf.at[b])
                if p == 0 and n_scalar == 2:
                    @pl.when(c == 1)
                    def _():
                        pltpu.sync_copy(srcp_r.at[n2, pl.ds(rb, KC)],
                                        rbuf.at[b])
                        for j in range(KC):
                            pltpu.async_copy(
                                hcol_r.at[rbuf.at[b, j]],
                                cbuf.at[b, pl.ds(j * LN, LN)], semc[b])

            def process(i, b):
                pltpu.make_async_copy(
                    htab_r.at[pl.ds(0, KC * LN)], rows.at[b],
                    semg[b]).wait()
                for j in range(KC):
                    pltpu.sync_copy(rows.at[b, pl.ds(j * LN, LN)],
                                    acc.at[dbuf.at[b, j]], add=True)
                if p == 0 and n_scalar >= 1:
                    @pl.when(c == 0)
                    def _():
                        for j in range(KC):
                            pltpu.sync_copy(gbuf.at[b, j],
                                            accs.at[dbuf.at[b, j]], add=True)
                if p == 0 and n_scalar == 2:
                    @pl.when(c == 1)
                    def _():
                        pltpu.make_async_copy(
                            hcol_r.at[pl.ds(0, KC * LN)], cbuf.at[b],
                            semc[b]).wait()
                        for j in range(KC):
                            pltpu.sync_copy(
                                cbuf.at[b, pl.ds(j * LN, LN)],
                                accs.at[dbuf.at[b, j]], add=True)

                @pl.when(i + 2 < n_chunks)
                def _():
                    load(i + 2, b)

            load(0, 0)
            load(1, 1)
            pltpu.sync_copy(zrow_r, acc.at[pl.ds(s * rpt, rpt)])
            if p == 0 and n_scalar >= 1:
                @pl.when(c < n_scalar)
                def _():
                    pltpu.sync_copy(zsc_r, accs.at[pl.ds(s * rpt, rpt)])
            plsc.subcore_barrier()

            def chunk_pair(g, carry):
                process(2 * g, 0)
                process(2 * g + 1, 1)
                return carry

            lax.fori_loop(0, n_chunks // 2, chunk_pair, 0)
            plsc.subcore_barrier()
            pltpu.sync_copy(acc.at[pl.ds(s * rpt, rpt)],
                            outh_r.at[blk, pl.ds(s * rpt, rpt)])
            if p == 0 and n_scalar >= 1:
                @pl.when(c < n_scalar)
                def _():
                    pltpu.sync_copy(accs.at[pl.ds(s * rpt, rpt)],
                                    outs_r.at[c, pl.ds(s * rpt, rpt)])
            plsc.subcore_barrier()

    return body(htab, dst2, srcp, gv2, hcol, zrow, zsc)


def _sc_layer(h, dst2, srcr, gv2, n_pad, ep_rows):
    n, d = h.shape
    n2 = max(2, (d // CB) - ((d // CB) % 2))
    rem = d - n2 * CB
    n_scalar = 1 + (1 if rem else 0)
    htab = (h[:, :n2 * CB].reshape(n, n2, CB).transpose(1, 0, 2)
            .reshape(n2 * n, CB))
    offs = [srcr + b * n for b in range(n2)]
    if rem:
        offs.append(srcr)
        hcol = h[:, n2 * CB]
    else:
        hcol = jnp.zeros((8,), jnp.float32)
    srcp = jnp.stack(offs).reshape(len(offs), ep_rows, LN)
    rpt = n_pad // NS
    zrow = jnp.zeros((rpt, CB), jnp.float32)
    zsc = jnp.zeros((rpt,), jnp.float32)
    outh, outs = _sc_aggregate(htab, dst2, srcp, gv2, hcol, zrow, zsc,
                               n2, n_pad, n_scalar, ep_rows)
    hag = outh.transpose(1, 0, 2).reshape(n_pad, n2 * CB)[:n]
    cols = [hag]
    if rem:
        cols.append(outs[1, :n, None])
    cols.append(outs[0, :n, None])
    return jnp.concatenate(cols, axis=1)


_TC_R = 512


def _stats_body(agg_ref, out_ref):
    i = pl.program_id(0)
    x = agg_ref[...]
    s1 = jnp.sum(x, axis=0)
    s2 = jnp.sum(x * x, axis=0)
    part = jnp.concatenate(
        [s1[None], s2[None], jnp.zeros((6, x.shape[1]), jnp.float32)], axis=0)

    @pl.when(i == 0)
    def _():
        out_ref[...] = part

    @pl.when(i > 0)
    def _():
        out_ref[...] += part


def _make_transform_body(n):
    inv_n = 1.0 / n

    def body(agg_ref, h_ref, st_ref, w1_ref, w2a_ref, w2b_ref, o_ref):
        st = st_ref[...]
        mean = st[0:1] * inv_n
        var = st[1:2] * inv_n - mean * mean
        inv = lax.rsqrt(var + EPS)
        bn = (agg_ref[...] - mean) * inv
        r = jnp.maximum(
            jnp.dot(bn, w1_ref[...], preferred_element_type=jnp.float32), 0.0)
        o_ref[...] = (
            jnp.dot(h_ref[...], w2a_ref[...],
                    preferred_element_type=jnp.float32)
            + jnp.dot(r, w2b_ref[...], preferred_element_type=jnp.float32))

    return body


def _tc_dense(agg, h, W1, W2):
    n, da = agg.shape
    di = h.shape[1]
    do = W2.shape[0]
    dap, dip, dop = _rup(da, 128), _rup(di, 128), _rup(do, 128)
    nt = _rup(n, _TC_R)
    aggp = jnp.zeros((nt, dap), jnp.float32).at[:n, :da].set(agg)
    hp = jnp.zeros((nt, dip), jnp.float32).at[:n, :di].set(h)
    w1t = jnp.zeros((dap, dap), jnp.float32).at[:da, :da].set(W1.T)
    w2at = jnp.zeros((dip, dop), jnp.float32).at[:di, :do].set(W2[:, :di].T)
    w2bt = jnp.zeros((dap, dop), jnp.float32).at[:da, :do].set(W2[:, di:].T)
    nb = nt // _TC_R
    stats = pl.pallas_call(
        _stats_body,
        grid=(nb,),
        in_specs=[pl.BlockSpec((_TC_R, dap), lambda i: (i, 0))],
        out_specs=pl.BlockSpec((8, dap), lambda i: (0, 0)),
        out_shape=jax.ShapeDtypeStruct((8, dap), jnp.float32),
    )(aggp)
    out = pl.pallas_call(
        _make_transform_body(n),
        grid=(nb,),
        in_specs=[
            pl.BlockSpec((_TC_R, dap), lambda i: (i, 0)),
            pl.BlockSpec((_TC_R, dip), lambda i: (i, 0)),
            pl.BlockSpec((8, dap), lambda i: (0, 0)),
            pl.BlockSpec((dap, dap), lambda i: (0, 0)),
            pl.BlockSpec((dip, dop), lambda i: (0, 0)),
            pl.BlockSpec((dap, dop), lambda i: (0, 0)),
        ],
        out_specs=pl.BlockSpec((_TC_R, dop), lambda i: (i, 0)),
        out_shape=jax.ShapeDtypeStruct((nt, dop), jnp.float32),
    )(aggp, hp, stats, w1t, w2at, w2bt)
    return out[:n, :do]


def kernel(u, v, t, event, h0, W1_0, W2_0, W1_1, W2_1):
    e = u.shape[0]
    n = h0.shape[0]
    tfirst = t[0]
    tlast = t[event - 1]
    g = (tlast - t) / (1.0 + tlast - tfirst)

    per = NS * KC * LN * 2
    ep = _rup(2 * e, per)
    ep_rows = ep // LN
    n_pad = _rup(n, NS * 8)
    if ep > 2 * e and n_pad == n:
        n_pad += NS * 8
    pad = ep - 2 * e
    dst = jnp.concatenate(
        [u.astype(jnp.int32), v.astype(jnp.int32),
         jnp.full((pad,), n_pad - 1, jnp.int32)])
    srcr = jnp.concatenate(
        [v.astype(jnp.int32), u.astype(jnp.int32),
         jnp.zeros((pad,), jnp.int32)])
    gv2 = jnp.concatenate([g, g, jnp.zeros((pad,), jnp.float32)])
    dst2 = dst.reshape(ep_rows, LN)
    gv2 = gv2.reshape(ep_rows, LN)

    h = h0
    hs = [h]
    for W1, W2 in ((W1_0, W2_0), (W1_1, W2_1)):
        agg = _sc_layer(h, dst2, srcr, gv2, n_pad, ep_rows)
        h = _tc_dense(agg, h, W1, W2)
        hs.append(h)
    return tuple(hs)


# speedup vs baseline: 5.5484x; 1.1233x over previous
import functools

import jax
import jax.numpy as jnp
from jax import lax
from jax.experimental import pallas as pl
from jax.experimental.pallas import tpu as pltpu
from jax.experimental.pallas import tpu_sc as plsc

EPS = 1e-5
NC = 2
NS = 16
LN = 128
KC = 2
CB = 32


def _rup(x, m):
    return ((x + m - 1) // m) * m


def _sc_aggregate(htab, dst2, srcp, gv2, hcol, zrow, zsc,
                  n2, n_pad, col_mode, ep_rows):
    rpt = n_pad // NS
    t_rows = ep_rows // NS
    n_pass = n2 // 2
    n_chunks = t_rows // KC
    assert n_chunks % 2 == 0
    mesh = plsc.VectorSubcoreMesh(core_axis_name="c", subcore_axis_name="s")

    @functools.partial(
        pl.kernel,
        out_type=(
            jax.ShapeDtypeStruct((n2, n_pad, CB), jnp.float32),
            jax.ShapeDtypeStruct((2, n_pad), jnp.float32),
        ),
        mesh=mesh,
        compiler_params=pltpu.CompilerParams(use_tc_tiling_on_sc=False),
        scratch_types=(
            pltpu.VMEM_SHARED((n_pad, CB), jnp.float32),
            pltpu.VMEM_SHARED((n_pad,), jnp.float32),
            pltpu.VMEM((2, KC, LN), jnp.int32),
            pltpu.VMEM((2, KC, LN), jnp.int32),
            pltpu.VMEM((2, KC, LN), jnp.int32),
            pltpu.VMEM((2, KC * LN, CB), jnp.float32),
            pltpu.VMEM((2, KC, LN), jnp.float32),
            pltpu.VMEM((2, KC * LN), jnp.float32),
            pltpu.SemaphoreType.DMA,
            pltpu.SemaphoreType.DMA,
            pltpu.SemaphoreType.DMA,
            pltpu.SemaphoreType.DMA,
        ),
    )
    def body(htab_r, dst_r, srcp_r, gv_r, hcol_r, zrow_r, zsc_r,
             outh_r, outs_r,
             acc, accs, dbuf, sbuf, rbuf, rows, gbuf, cbuf,
             semg0, semg1, semc0, semc1):
        c = lax.axis_index("c")
        s = lax.axis_index("s")
        semg = (semg0, semg1)
        semc = (semc0, semc1)
        for p in range(n_pass):
            blk = p * 2 + c

            def load(i, b):
                rb = s * t_rows + i * KC
                pltpu.sync_copy(dst_r.at[pl.ds(rb, KC)], dbuf.at[b])
                pltpu.sync_copy(srcp_r.at[blk, pl.ds(rb, KC)], sbuf.at[b])
                for j in range(KC):
                    pltpu.async_copy(
                        htab_r.at[sbuf.at[b, j]],
                        rows.at[b, pl.ds(j * LN, LN)], semg[b])
                if p == 0:
                    if not col_mode:
                        @pl.when(c == b)
                        def _():
                            pltpu.sync_copy(gv_r.at[pl.ds(rb, KC)],
                                            gbuf.at[b])
                    else:
                        @pl.when(c == b)
                        def _():
                            pltpu.sync_copy(srcp_r.at[n2, pl.ds(rb, KC)],
                                            rbuf.at[b])
                            for j in range(KC):
                                pltpu.async_copy(
                                    hcol_r.at[rbuf.at[b, j]],
                                    cbuf.at[b, pl.ds(j * LN, LN)], semc[b])

            def process(i, b):
                pltpu.make_async_copy(
                    htab_r.at[pl.ds(0, KC * LN)], rows.at[b],
                    semg[b]).wait()
                for j in range(KC):
                    pltpu.sync_copy(rows.at[b, pl.ds(j * LN, LN)],
                                    acc.at[dbuf.at[b, j]], add=True)
                if p == 0:
                    if not col_mode:
                        @pl.when(c == b)
                        def _():
                            for j in range(KC):
                                pltpu.sync_copy(gbuf.at[b, j],
                                                accs.at[dbuf.at[b, j]],
                                                add=True)
                    else:
                        @pl.when(c == b)
                        def _():
                            pltpu.make_async_copy(
                                hcol_r.at[pl.ds(0, KC * LN)], cbuf.at[b],
                                semc[b]).wait()
                            for j in range(KC):
                                pltpu.sync_copy(
                                    cbuf.at[b, pl.ds(j * LN, LN)],
                                    accs.at[dbuf.at[b, j]], add=True)

                @pl.when(i + 2 < n_chunks)
                def _():
                    load(i + 2, b)

            load(0, 0)
            load(1, 1)
            pltpu.sync_copy(zrow_r, acc.at[pl.ds(s * rpt, rpt)])
            if p == 0:
                pltpu.sync_copy(zsc_r, accs.at[pl.ds(s * rpt, rpt)])
            plsc.subcore_barrier()

            def chunk_pair(gi, carry):
                process(2 * gi, 0)
                process(2 * gi + 1, 1)
                return carry

            lax.fori_loop(0, n_chunks // 2, chunk_pair, 0)
            plsc.subcore_barrier()
            pltpu.sync_copy(acc.at[pl.ds(s * rpt, rpt)],
                            outh_r.at[blk, pl.ds(s * rpt, rpt)])
            if p == 0:
                pltpu.sync_copy(accs.at[pl.ds(s * rpt, rpt)],
                                outs_r.at[c, pl.ds(s * rpt, rpt)])
            plsc.subcore_barrier()

    return body(htab, dst2, srcp, gv2, hcol, zrow, zsc)


_TC_R = 512


def _make_stats_body(n, n2):
    def body(ablk_ref, asc_ref, stb_ref, sts_ref):
        i = pl.program_id(0)
        lim = n - i * _TC_R
        m = (lax.broadcasted_iota(jnp.int32, (_TC_R, CB), 0)
             < lim).astype(jnp.float32)
        x = ablk_ref[...] * m[None]
        xs = asc_ref[...] * m
        s1 = jnp.sum(x, axis=1)
        s2 = jnp.sum(x * x, axis=1)
        pb = jnp.concatenate(
            [s1[:, None], s2[:, None],
             jnp.zeros((n2, 6, CB), jnp.float32)], axis=1)
        ss1 = jnp.sum(xs, axis=0)
        ss2 = jnp.sum(xs * xs, axis=0)
        ps = jnp.concatenate(
            [ss1[None], ss2[None], jnp.zeros((6, CB), jnp.float32)], axis=0)

        @pl.when(i == 0)
        def _():
            stb_ref[...] = pb
            sts_ref[...] = ps

        @pl.when(i > 0)
        def _():
            stb_ref[...] += pb
            sts_ref[...] += ps

    return body


def _make_transform_body(n, n2, n_blk_out):
    inv_n = 1.0 / n

    def body(ablk_ref, asc_ref, hin_ref, stb_ref, sts_ref,
             w1b_ref, w1s_ref, w2a_ref, w2b_ref, *out_refs):
        stb = stb_ref[...]
        mb = stb[:, 0:1] * inv_n
        vb = stb[:, 1:2] * inv_n - mb * mb
        ib = lax.rsqrt(vb + EPS)
        xb = (ablk_ref[...] - mb) * ib
        sts = sts_ref[...]
        ms = sts[0:1] * inv_n
        vs = sts[1:2] * inv_n - ms * ms
        ivs = lax.rsqrt(vs + EPS)
        xs = (asc_ref[...] - ms) * ivs
        w1b = w1b_ref[...]
        r = jnp.dot(xs, w1s_ref[...], preferred_element_type=jnp.float32)
        for b in range(n2):
            r = r + jnp.dot(xb[b], w1b[b],
                            preferred_element_type=jnp.float32)
        r = jnp.maximum(r, 0.0)
        o = (jnp.dot(hin_ref[...], w2a_ref[...],
                     preferred_element_type=jnp.float32)
             + jnp.dot(r, w2b_ref[...], preferred_element_type=jnp.float32))
        out_refs[0][...] = o
        if n_blk_out:
            ob = out_refs[1]
            for b in range(n_blk_out):
                ob[b] = o[:, b * CB:(b + 1) * CB]

    return body


def _tc_dense(ablk, asc, hin, W1, W2, n, di, n_blk_out):
    n2, n_pad, _ = ablk.shape
    da = W1.shape[0]
    do = W2.shape[0]
    dip = hin.shape[1]
    dhp = _rup(da, 128)
    dop = _rup(do, 128)
    nsc = da - n2 * CB
    w1t = W1.T
    w1b = (jnp.zeros((n2, CB, dhp), jnp.float32)
           .at[:, :, :da].set(w1t[:n2 * CB].reshape(n2, CB, da)))
    w1s = jnp.zeros((CB, dhp), jnp.float32).at[:nsc, :da].set(w1t[n2 * CB:])
    w2a = jnp.zeros((dip, dop), jnp.float32).at[:di, :do].set(W2[:, :di].T)
    w2b = jnp.zeros((dhp, dop), jnp.float32).at[:da, :do].set(W2[:, di:].T)
    nb = n_pad // _TC_R
    stb, sts = pl.pallas_call(
        _make_stats_body(n, n2),
        grid=(nb,),
        in_specs=[
            pl.BlockSpec((n2, _TC_R, CB), lambda i: (0, i, 0)),
            pl.BlockSpec((_TC_R, CB), lambda i: (i, 0)),
        ],
        out_specs=[
            pl.BlockSpec((n2, 8, CB), lambda i: (0, 0, 0)),
            pl.BlockSpec((8, CB), lambda i: (0, 0)),
        ],
        out_shape=(
            jax.ShapeDtypeStruct((n2, 8, CB), jnp.float32),
            jax.ShapeDtypeStruct((8, CB), jnp.float32),
        ),
    )(ablk, asc)
    out_specs = [pl.BlockSpec((_TC_R, dop), lambda i: (i, 0))]
    out_shape = [jax.ShapeDtypeStruct((n_pad, dop), jnp.float32)]
    if n_blk_out:
        out_specs.append(
            pl.BlockSpec((n_blk_out, _TC_R, CB), lambda i: (0, i, 0)))
        out_shape.append(
            jax.ShapeDtypeStruct((n_blk_out, n_pad, CB), jnp.float32))
    outs = pl.pallas_call(
        _make_transform_body(n, n2, n_blk_out),
        grid=(nb,),
        in_specs=[
            pl.BlockSpec((n2, _TC_R, CB), lambda i: (0, i, 0)),
            pl.BlockSpec((_TC_R, CB), lambda i: (i, 0)),
            pl.BlockSpec((_TC_R, dip), lambda i: (i, 0)),
            pl.BlockSpec((n2, 8, CB), lambda i: (0, 0, 0)),
            pl.BlockSpec((8, CB), lambda i: (0, 0)),
            pl.BlockSpec((n2, CB, dhp), lambda i: (0, 0, 0)),
            pl.BlockSpec((CB, dhp), lambda i: (0, 0)),
            pl.BlockSpec((dip, dop), lambda i: (0, 0)),
            pl.BlockSpec((dhp, dop), lambda i: (0, 0)),
        ],
        out_specs=out_specs,
        out_shape=tuple(out_shape),
    )(ablk, asc, hin, stb, sts, w1b, w1s, w2a, w2b)
    if n_blk_out:
        return outs
    return outs[0], None


def kernel(u, v, t, event, h0, W1_0, W2_0, W1_1, W2_1):
    e = u.shape[0]
    n = h0.shape[0]
    d0 = h0.shape[1]
    tfirst = t[0]
    tlast = t[event - 1]
    g = (tlast - t) / (1.0 + tlast - tfirst)

    per = NS * KC * LN * 2
    ep = _rup(2 * e, per)
    ep_rows = ep // LN
    n_pad = _rup(n + 1, _TC_R)
    pad = ep - 2 * e
    dst2 = jnp.concatenate(
        [u.astype(jnp.int32), v.astype(jnp.int32),
         jnp.full((pad,), n_pad - 1, jnp.int32)]).reshape(ep_rows, LN)
    srcr = jnp.concatenate(
        [v.astype(jnp.int32), u.astype(jnp.int32),
         jnp.zeros((pad,), jnp.int32)])
    gv2 = jnp.concatenate(
        [g, g, jnp.zeros((pad,), jnp.float32)]).reshape(ep_rows, LN)

    rpt = n_pad // NS
    zrow = jnp.zeros((rpt, CB), jnp.float32)
    zsc = jnp.zeros((rpt,), jnp.float32)
    dcol = jnp.zeros((8,), jnp.float32)
    do0 = W2_0.shape[0]
    do1 = W2_1.shape[0]
    nb0 = d0 // CB
    nb1 = do0 // CB

    htab0 = (jnp.zeros((nb0, n_pad, CB), jnp.float32)
             .at[:, :n].set(h0.reshape(n, nb0, CB).transpose(1, 0, 2))
             .reshape(nb0 * n_pad, CB))
    srcp0 = (srcr[None, :]
             + (jnp.arange(nb0, dtype=jnp.int32) * n_pad)[:, None]
             ).reshape(nb0, ep_rows, LN)
    outh0, outs0 = _sc_aggregate(htab0, dst2, srcp0, gv2, dcol, zrow, zsc,
                                 nb0, n_pad, False, ep_rows)
    gagg = outs0[0] + outs0[1]
    asc0 = jnp.zeros((n_pad, CB), jnp.float32).at[:, 0].set(gagg)
    h0p = jnp.zeros((n_pad, _rup(d0, 128)), jnp.float32).at[:n, :d0].set(h0)
    h1n, h1blk = _tc_dense(outh0, asc0, h0p, W1_0, W2_0, n, d0, nb1)

    htab1 = h1blk.reshape(nb1 * n_pad, CB)
    hcol1 = h1n[:, nb1 * CB]
    srcp1 = jnp.concatenate(
        [srcr[None, :]
         + (jnp.arange(nb1, dtype=jnp.int32) * n_pad)[:, None],
         srcr[None, :]], axis=0).reshape(nb1 + 1, ep_rows, LN)
    outh1, outs1 = _sc_aggregate(htab1, dst2, srcp1, gv2, hcol1, zrow, zsc,
                                 nb1, n_pad, True, ep_rows)
    asc1 = (jnp.zeros((n_pad, CB), jnp.float32)
            .at[:, 0].set(outs1[0] + outs1[1])
            .at[:, 1].set(gagg))
    h2n, _ = _tc_dense(outh1, asc1, h1n, W1_1, W2_1, n, do0, 0)

    return (h0, h1n[:n, :do0], h2n[:n, :do1])


# speedup vs baseline: 6.3681x; 1.1477x over previous
import functools

import jax
import jax.numpy as jnp
from jax import lax
from jax.experimental import pallas as pl
from jax.experimental.pallas import tpu as pltpu
from jax.experimental.pallas import tpu_sc as plsc

EPS = 1e-5
NC = 2
NS = 16
LN = 128
KC = 2
CB = 32


def _rup(x, m):
    return ((x + m - 1) // m) * m


def _sc_aggregate(htab, dst2, srcp, gv2, hcol, zrow, zsc,
                  n2, n_pad, col_mode, ep_rows):
    rpt = n_pad // NS
    t_rows = ep_rows // NS
    n_pass = n2 // 2
    n_chunks = t_rows // KC
    assert n_chunks % 2 == 0
    mesh = plsc.VectorSubcoreMesh(core_axis_name="c", subcore_axis_name="s")

    @functools.partial(
        pl.kernel,
        out_type=(
            jax.ShapeDtypeStruct((n2, n_pad, CB), jnp.float32),
            jax.ShapeDtypeStruct((2, n_pad), jnp.float32),
        ),
        mesh=mesh,
        compiler_params=pltpu.CompilerParams(use_tc_tiling_on_sc=False),
        scratch_types=(
            pltpu.VMEM_SHARED((n_pad, CB), jnp.float32),
            pltpu.VMEM_SHARED((n_pad,), jnp.float32),
            pltpu.VMEM((2, KC, LN), jnp.int32),
            pltpu.VMEM((2, KC, LN), jnp.int32),
            pltpu.VMEM((2, KC, LN), jnp.int32),
            pltpu.VMEM((2, KC * LN, CB), jnp.float32),
            pltpu.VMEM((2, KC, LN), jnp.float32),
            pltpu.VMEM((2, KC * LN), jnp.float32),
            pltpu.SemaphoreType.DMA,
            pltpu.SemaphoreType.DMA,
            pltpu.SemaphoreType.DMA,
            pltpu.SemaphoreType.DMA,
        ),
    )
    def body(htab_r, dst_r, srcp_r, gv_r, hcol_r, zrow_r, zsc_r,
             outh_r, outs_r,
             acc, accs, dbuf, sbuf, rbuf, rows, gbuf, cbuf,
             semg0, semg1, semc0, semc1):
        c = lax.axis_index("c")
        s = lax.axis_index("s")
        semg = (semg0, semg1)
        semc = (semc0, semc1)
        for p in range(n_pass):
            blk = p * 2 + c

            def load(i, b):
                rb = s * t_rows + i * KC
                pltpu.sync_copy(dst_r.at[pl.ds(rb, KC)], dbuf.at[b])
                pltpu.sync_copy(srcp_r.at[blk, pl.ds(rb, KC)], sbuf.at[b])
                for j in range(KC):
                    pltpu.async_copy(
                        htab_r.at[sbuf.at[b, j]],
                        rows.at[b, pl.ds(j * LN, LN)], semg[b])
                if p == 0:
                    if not col_mode:
                        @pl.when(c == b)
                        def _():
                            pltpu.sync_copy(gv_r.at[pl.ds(rb, KC)],
                                            gbuf.at[b])
                    else:
                        @pl.when(c == b)
                        def _():
                            pltpu.sync_copy(srcp_r.at[n2, pl.ds(rb, KC)],
                                            rbuf.at[b])
                            for j in range(KC):
                                pltpu.async_copy(
                                    hcol_r.at[rbuf.at[b, j]],
                                    cbuf.at[b, pl.ds(j * LN, LN)], semc[b])

            def process(i, b):
                pltpu.make_async_copy(
                    htab_r.at[pl.ds(0, KC * LN)], rows.at[b],
                    semg[b]).wait()
                for j in range(KC):
                    pltpu.sync_copy(rows.at[b, pl.ds(j * LN, LN)],
                                    acc.at[dbuf.at[b, j]], add=True)
                if p == 0:
                    if not col_mode:
                        @pl.when(c == b)
                        def _():
                            for j in range(KC):
                                pltpu.sync_copy(gbuf.at[b, j],
                                                accs.at[dbuf.at[b, j]],
                                                add=True)
                    else:
                        @pl.when(c == b)
                        def _():
                            pltpu.make_async_copy(
                                hcol_r.at[pl.ds(0, KC * LN)], cbuf.at[b],
                                semc[b]).wait()
                            for j in range(KC):
                                pltpu.sync_copy(
                                    cbuf.at[b, pl.ds(j * LN, LN)],
                                    accs.at[dbuf.at[b, j]], add=True)

                @pl.when(i + 2 < n_chunks)
                def _():
                    load(i + 2, b)

            load(0, 0)
            load(1, 1)
            pltpu.sync_copy(zrow_r, acc.at[pl.ds(s * rpt, rpt)])
            if p == 0:
                pltpu.sync_copy(zsc_r, accs.at[pl.ds(s * rpt, rpt)])
            plsc.subcore_barrier()

            def chunk_pair(gi, carry):
                process(2 * gi, 0)
                process(2 * gi + 1, 1)
                return carry

            lax.fori_loop(0, n_chunks // 2, chunk_pair, 0)
            plsc.subcore_barrier()
            pltpu.sync_copy(acc.at[pl.ds(s * rpt, rpt)],
                            outh_r.at[blk, pl.ds(s * rpt, rpt)])
            if p == 0:
                pltpu.sync_copy(accs.at[pl.ds(s * rpt, rpt)],
                                outs_r.at[c, pl.ds(s * rpt, rpt)])
            plsc.subcore_barrier()

    return body(htab, dst2, srcp, gv2, hcol, zrow, zsc)


_TC_R = 512


def _make_stats_body(n, n2, k):
    def body(ablk_ref, scp_ref, stb_ref, sts_ref):
        i = pl.program_id(0)
        lim = n - i * _TC_R
        m = (lax.broadcasted_iota(jnp.int32, (_TC_R, CB), 0)
             < lim).astype(jnp.float32)
        x = ablk_ref[...] * m[None]
        mr = (lax.broadcasted_iota(jnp.int32, (1, _TC_R), 1)
              < lim).astype(jnp.float32)
        xp = scp_ref[...] * mr
        xs = xp.reshape(k, 2, _TC_R).sum(axis=1)
        s1 = jnp.sum(x, axis=1)
        s2 = jnp.sum(x * x, axis=1)
        pb = jnp.concatenate(
            [s1[:, None], s2[:, None],
             jnp.zeros((n2, 6, CB), jnp.float32)], axis=1)
        ps = jnp.concatenate(
            [jnp.sum(xs, axis=1, keepdims=True),
             jnp.sum(xs * xs, axis=1, keepdims=True),
             jnp.zeros((k, 6), jnp.float32)], axis=1)

        @pl.when(i == 0)
        def _():
            stb_ref[...] = pb
            sts_ref[...] = ps

        @pl.when(i > 0)
        def _():
            stb_ref[...] += pb
            sts_ref[...] += ps

    return body


def _make_transform_body(n, n2, k, do, n_blk_out):
    inv_n = 1.0 / n

    def body(ablk_ref, scp_ref, hin_ref, stb_ref, sts_ref,
             w1b_ref, w1s_ref, w2a_ref, w2b_ref, *out_refs):
        stb = stb_ref[...]
        mb = stb[:, 0:1] * inv_n
        vb = stb[:, 1:2] * inv_n - mb * mb
        ib = lax.rsqrt(vb + EPS)
        xb = (ablk_ref[...] - mb) * ib
        xs = scp_ref[...].reshape(k, 2, _TC_R).sum(axis=1)
        sts = sts_ref[...]
        ms = sts[:, 0:1] * inv_n
        vs = sts[:, 1:2] * inv_n - ms * ms
        ivs = lax.rsqrt(vs + EPS)
        bnp = (xs - ms) * ivs
        w1b = w1b_ref[...]
        r = lax.dot_general(
            bnp, w1s_ref[...], (((0,), (0,)), ((), ())),
            preferred_element_type=jnp.float32)
        for b in range(n2):
            r = r + jnp.dot(xb[b], w1b[b],
                            preferred_element_type=jnp.float32)
        r = jnp.maximum(r, 0.0)
        o = (jnp.dot(hin_ref[...], w2a_ref[...],
                     preferred_element_type=jnp.float32)
             + jnp.dot(r, w2b_ref[...], preferred_element_type=jnp.float32))
        out_refs[0][...] = o[:, :do]
        if n_blk_out:
            out_refs[1][...] = o
            out_refs[2][...] = o[:, :n_blk_out * CB]

    return body


def _tc_dense(ablk, scp, hin, W1, W2, n, di, n_blk_out):
    n2, n_pad, _ = ablk.shape
    k = scp.shape[0] // 2
    da = W1.shape[0]
    do = W2.shape[0]
    dip = hin.shape[1]
    dhp = _rup(da, 128)
    dop = _rup(do, 128)
    w1t = W1.T
    w1b = (jnp.zeros((n2, CB, dhp), jnp.float32)
           .at[:, :, :da].set(w1t[:n2 * CB].reshape(n2, CB, da)))
    w1s = jnp.zeros((k, dhp), jnp.float32).at[:, :da].set(w1t[n2 * CB:])
    w2a = jnp.zeros((dip, dop), jnp.float32).at[:di, :do].set(W2[:, :di].T)
    w2b = jnp.zeros((dhp, dop), jnp.float32).at[:da, :do].set(W2[:, di:].T)
    nb = n_pad // _TC_R
    stb, sts = pl.pallas_call(
        _make_stats_body(n, n2, k),
        grid=(nb,),
        in_specs=[
            pl.BlockSpec((n2, _TC_R, CB), lambda i: (0, i, 0)),
            pl.BlockSpec((2 * k, _TC_R), lambda i: (0, i)),
        ],
        out_specs=[
            pl.BlockSpec((n2, 8, CB), lambda i: (0, 0, 0)),
            pl.BlockSpec((k, 8), lambda i: (0, 0)),
        ],
        out_shape=(
            jax.ShapeDtypeStruct((n2, 8, CB), jnp.float32),
            jax.ShapeDtypeStruct((k, 8), jnp.float32),
        ),
    )(ablk, scp)
    out_specs = [pl.BlockSpec((_TC_R, do), lambda i: (i, 0))]
    out_shape = [jax.ShapeDtypeStruct((n, do), jnp.float32)]
    if n_blk_out:
        out_specs.append(pl.BlockSpec((_TC_R, dop), lambda i: (i, 0)))
        out_shape.append(jax.ShapeDtypeStruct((n_pad, dop), jnp.float32))
        out_specs.append(
            pl.BlockSpec((_TC_R, n_blk_out * CB), lambda i: (i, 0)))
        out_shape.append(
            jax.ShapeDtypeStruct((n_pad, n_blk_out * CB), jnp.float32))
    outs = pl.pallas_call(
        _make_transform_body(n, n2, k, do, n_blk_out),
        grid=(nb,),
        in_specs=[
            pl.BlockSpec((n2, _TC_R, CB), lambda i: (0, i, 0)),
            pl.BlockSpec((2 * k, _TC_R), lambda i: (0, i)),
            pl.BlockSpec((_TC_R, dip), lambda i: (i, 0)),
            pl.BlockSpec((n2, 8, CB), lambda i: (0, 0, 0)),
            pl.BlockSpec((k, 8), lambda i: (0, 0)),
            pl.BlockSpec((n2, CB, dhp), lambda i: (0, 0, 0)),
            pl.BlockSpec((k, dhp), lambda i: (0, 0)),
            pl.BlockSpec((dip, dop), lambda i: (0, 0)),
            pl.BlockSpec((dhp, dop), lambda i: (0, 0)),
        ],
        out_specs=out_specs,
        out_shape=tuple(out_shape),
    )(ablk, scp, hin, stb, sts, w1b, w1s, w2a, w2b)
    if n_blk_out:
        return outs
    return outs[0], None, None


def kernel(u, v, t, event, h0, W1_0, W2_0, W1_1, W2_1):
    e = u.shape[0]
    n = h0.shape[0]
    d0 = h0.shape[1]
    tfirst = t[0]
    tlast = t[event - 1]
    g = (tlast - t) / (1.0 + tlast - tfirst)

    per = NS * KC * LN * 2
    ep = _rup(2 * e, per)
    ep_rows = ep // LN
    n_pad = _rup(n + 1, _TC_R)
    pad = ep - 2 * e
    dst2 = jnp.concatenate(
        [u.astype(jnp.int32), v.astype(jnp.int32),
         jnp.full((pad,), n_pad - 1, jnp.int32)]).reshape(ep_rows, LN)
    srcr = jnp.concatenate(
        [v.astype(jnp.int32), u.astype(jnp.int32),
         jnp.zeros((pad,), jnp.int32)])
    gv2 = jnp.concatenate(
        [g, g, jnp.zeros((pad,), jnp.float32)]).reshape(ep_rows, LN)

    rpt = n_pad // NS
    zrow = jnp.zeros((rpt, CB), jnp.float32)
    zsc = jnp.zeros((rpt,), jnp.float32)
    dcol = jnp.zeros((8,), jnp.float32)
    do0 = W2_0.shape[0]
    nb0 = d0 // CB
    nb1 = do0 // CB

    htab0 = (jnp.zeros((nb0, n_pad, CB), jnp.float32)
             .at[:, :n].set(h0.reshape(n, nb0, CB).transpose(1, 0, 2))
             .reshape(nb0 * n_pad, CB))
    srcp0 = (srcr[None, :]
             + (jnp.arange(nb0, dtype=jnp.int32) * n_pad)[:, None]
             ).reshape(nb0, ep_rows, LN)
    outh0, outs0 = _sc_aggregate(htab0, dst2, srcp0, gv2, dcol, zrow, zsc,
                                 nb0, n_pad, False, ep_rows)
    h0p = jnp.zeros((n_pad, _rup(d0, 128)), jnp.float32).at[:n, :d0].set(h0)
    h1x, h1p, h1d = _tc_dense(outh0, outs0, h0p, W1_0, W2_0, n, d0, nb1)

    htab1 = h1d.reshape(nb1 * n_pad, CB)
    hcol1 = h1p[:, nb1 * CB]
    srcp1 = jnp.concatenate(
        [srcr[None, :] * nb1
         + jnp.arange(nb1, dtype=jnp.int32)[:, None],
         srcr[None, :]], axis=0).reshape(nb1 + 1, ep_rows, LN)
    outh1, outs1 = _sc_aggregate(htab1, dst2, srcp1, gv2, hcol1, zrow, zsc,
                                 nb1, n_pad, True, ep_rows)
    scp1 = jnp.concatenate([outs1, outs0], axis=0)
    h2x, _, _ = _tc_dense(outh1, scp1, h1p, W1_1, W2_1, n, do0, 0)

    return (h0, h1x, h2x)
